# baseline (speedup 1.0000x reference)
import functools

import jax
import jax.numpy as jnp
from jax import lax
from jax.experimental import pallas as pl
from jax.experimental.pallas import tpu as pltpu
from jax.experimental.pallas import tpu_sc as plsc

MIN_POS_K = 0.1
MAX_POS_K = 1000.0
N_POS_EMB_K = 512
N_HEADS_K = 12

NC = 2
NS = 16
NW = NC * NS
LANES = 16

ROWB = 8
COLB = 256
PADV = 520
NPAIR = N_HEADS_K // 2


def _idx_body(d_ref, idx_ref):
    mn_log = jnp.log10(jnp.float32(MIN_POS_K))
    mx_log = jnp.log10(jnp.float32(MAX_POS_K))
    t = jnp.clip(d_ref[...], MIN_POS_K, MAX_POS_K)
    t = jnp.log10(t)
    t = (t - mn_log) / (mx_log - mn_log)
    t = N_POS_EMB_K * t
    idx_ref[...] = t.astype(jnp.int32)


def _compute_idx(d2):
    rows, cols = d2.shape
    br = 256
    return pl.pallas_call(
        _idx_body,
        grid=(rows // br,),
        in_specs=[pl.BlockSpec((br, cols), lambda i: (i, 0))],
        out_specs=pl.BlockSpec((br, cols), lambda i: (i, 0)),
        out_shape=jax.ShapeDtypeStruct((rows, cols), jnp.int32),
    )(d2)


def _sc_gather(idx2, tab_t):
    rows, cols = idx2.shape
    bands = rows // ROWB
    bands_per_w = bands // NW
    ncol = cols // COLB
    mesh = plsc.VectorSubcoreMesh(
        core_axis_name="c", subcore_axis_name="s", num_cores=NC, num_subcores=NS
    )

    nchunks = bands_per_w * ncol
    qper = COLB // LANES

    @functools.partial(
        pl.kernel,
        out_type=jax.ShapeDtypeStruct((1, N_HEADS_K, rows, cols), jnp.float32),
        mesh=mesh,
        compiler_params=pltpu.CompilerParams(needs_layout_passes=False),
        scratch_types=[
            pltpu.VMEM((NPAIR * PADV,), jnp.int32),
            pltpu.VMEM((ROWB, COLB), jnp.int32),
            pltpu.VMEM((ROWB, COLB), jnp.int32),
            pltpu.VMEM((N_HEADS_K, ROWB, COLB), jnp.float32),
            pltpu.VMEM((N_HEADS_K, ROWB, COLB), jnp.float32),
            pltpu.SemaphoreType.DMA,
            pltpu.SemaphoreType.DMA,
            pltpu.SemaphoreType.DMA,
            pltpu.SemaphoreType.DMA,
        ],
    )
    def run(idx_hbm, table_hbm, out_hbm, tab_v, idx_v0, idx_v1,
            rows_v0, rows_v1, sin0, sin1, sout0, sout1):
        wid = lax.axis_index("s") * NC + lax.axis_index("c")
        band0 = wid * bands_per_w
        idx_bufs = (idx_v0, idx_v1)
        rows_bufs = (rows_v0, rows_v1)
        sins = (sin0, sin1)
        souts = (sout0, sout1)
        pltpu.sync_copy(table_hbm, tab_v)

        def chunk_slices(k):
            r0 = pl.multiple_of((band0 + k // ncol) * ROWB, ROWB)
            c0 = pl.multiple_of((k % ncol) * COLB, COLB)
            return r0, c0

        def issue_in(k, p):
            r0, c0 = chunk_slices(k)
            pltpu.async_copy(
                idx_hbm.at[pl.ds(r0, ROWB), pl.ds(c0, COLB)], idx_bufs[p], sins[p]
            )

        def wait_in(p):
            pltpu.make_async_copy(
                idx_hbm.at[pl.ds(0, ROWB), pl.ds(0, COLB)], idx_bufs[p], sins[p]
            ).wait()

        def issue_outs(k, p):
            r0, c0 = chunk_slices(k)
            pltpu.async_copy(
                rows_bufs[p],
                out_hbm.at[0, :, pl.ds(r0, ROWB), pl.ds(c0, COLB)],
                souts[p],
            )

        def wait_outs(p):
            pltpu.make_async_copy(
                rows_bufs[p],
                out_hbm.at[0, :, pl.ds(0, ROWB), pl.ds(0, COLB)],
                souts[p],
            ).wait()

        def compute(p):
            idx_b, rows_b = idx_bufs[p], rows_bufs[p]

            @plsc.parallel_loop(0, ROWB * qper, unroll=4)
            def _(t):
                ri = t // qper
                q = t % qper
                f = idx_b[ri, pl.ds(q * LANES, LANES)]
                for hp in range(NPAIR):
                    w = plsc.load_gather(tab_v, [f + hp * PADV])
                    lo = plsc.bitcast(lax.shift_left(w, 16), jnp.float32)
                    hi = plsc.bitcast(w & jnp.int32(-65536), jnp.float32)
                    rows_b[2 * hp, ri, pl.ds(q * LANES, LANES)] = lo
                    rows_b[2 * hp + 1, ri, pl.ds(q * LANES, LANES)] = hi

        issue_in(0, 0)

        @pl.loop(0, nchunks // 2)
        def _(g):
            for p in range(2):
                k = g * 2 + p

                @pl.when(k + 1 < nchunks)
                def _():
                    issue_in(k + 1, 1 - p)

                wait_in(p)

                @pl.when(k >= 2)
                def _():
                    wait_outs(p)

                compute(p)
                issue_outs(k, p)

        wait_outs(0)
        wait_outs(1)

    return run(idx2, tab_t)


def kernel(d_mat, embeddings_table):
    b, rows, cols = d_mat.shape
    idx2 = _compute_idx(d_mat.reshape(b * rows, cols))
    tb = lax.bitcast_convert_type(
        embeddings_table.T.astype(jnp.bfloat16), jnp.uint16
    ).astype(jnp.uint32)
    packed = (tb[0::2, :] | (tb[1::2, :] << 16)).astype(jnp.int32)
    tab_t = (
        jnp.zeros((NPAIR, PADV), jnp.int32)
        .at[:, : N_POS_EMB_K + 1]
        .set(packed)
        .reshape(-1)
    )
    out = _sc_gather(idx2, tab_t)
    return out.transpose(0, 2, 3, 1)


# speedup vs baseline: 1.0299x; 1.0299x over previous
import functools

import jax
import jax.numpy as jnp
from jax import lax
from jax.experimental import pallas as pl
from jax.experimental.pallas import tpu as pltpu
from jax.experimental.pallas import tpu_sc as plsc

MIN_POS_K = 0.1
MAX_POS_K = 1000.0
N_POS_EMB_K = 512
N_HEADS_K = 12

NC = 2
NS = 16
NW = NC * NS
LANES = 16

ROWB = 8
COLB = 512
PADV = 520
NPAIR = N_HEADS_K // 2


def _idx_body(d_ref, idx_ref):
    mn_log = jnp.log10(jnp.float32(MIN_POS_K))
    mx_log = jnp.log10(jnp.float32(MAX_POS_K))
    t = jnp.clip(d_ref[...], MIN_POS_K, MAX_POS_K)
    t = jnp.log10(t)
    t = (t - mn_log) / (mx_log - mn_log)
    t = N_POS_EMB_K * t
    idx_ref[...] = t.astype(jnp.int32)


def _compute_idx(d2):
    rows, cols = d2.shape
    br = 256
    return pl.pallas_call(
        _idx_body,
        grid=(rows // br,),
        in_specs=[pl.BlockSpec((br, cols), lambda i: (i, 0))],
        out_specs=pl.BlockSpec((br, cols), lambda i: (i, 0)),
        out_shape=jax.ShapeDtypeStruct((rows, cols), jnp.int32),
    )(d2)


def _sc_gather(idx2, tab_t):
    rows, cols = idx2.shape
    bands = rows // ROWB
    bands_per_w = bands // NW
    ncol = cols // COLB
    mesh = plsc.VectorSubcoreMesh(
        core_axis_name="c", subcore_axis_name="s", num_cores=NC, num_subcores=NS
    )

    nchunks = bands_per_w * ncol
    qper = COLB // LANES

    @functools.partial(
        pl.kernel,
        out_type=jax.ShapeDtypeStruct((1, N_HEADS_K, rows, cols), jnp.float32),
        mesh=mesh,
        compiler_params=pltpu.CompilerParams(needs_layout_passes=False),
        scratch_types=[
            pltpu.VMEM((NPAIR * PADV,), jnp.int32),
            pltpu.VMEM((ROWB, COLB), jnp.int32),
            pltpu.VMEM((ROWB, COLB), jnp.int32),
            pltpu.VMEM((N_HEADS_K, ROWB, COLB), jnp.float32),
            pltpu.VMEM((N_HEADS_K, ROWB, COLB), jnp.float32),
            pltpu.SemaphoreType.DMA,
            pltpu.SemaphoreType.DMA,
            pltpu.SemaphoreType.DMA,
            pltpu.SemaphoreType.DMA,
        ],
    )
    def run(idx_hbm, table_hbm, out_hbm, tab_v, idx_v0, idx_v1,
            rows_v0, rows_v1, sin0, sin1, sout0, sout1):
        wid = lax.axis_index("s") * NC + lax.axis_index("c")
        band0 = wid * bands_per_w
        idx_bufs = (idx_v0, idx_v1)
        rows_bufs = (rows_v0, rows_v1)
        sins = (sin0, sin1)
        souts = (sout0, sout1)
        pltpu.sync_copy(table_hbm, tab_v)

        def chunk_slices(k):
            r0 = pl.multiple_of((band0 + k // ncol) * ROWB, ROWB)
            c0 = pl.multiple_of((k % ncol) * COLB, COLB)
            return r0, c0

        def issue_in(k, p):
            r0, c0 = chunk_slices(k)
            pltpu.async_copy(
                idx_hbm.at[pl.ds(r0, ROWB), pl.ds(c0, COLB)], idx_bufs[p], sins[p]
            )

        def wait_in(p):
            pltpu.make_async_copy(
                idx_hbm.at[pl.ds(0, ROWB), pl.ds(0, COLB)], idx_bufs[p], sins[p]
            ).wait()

        def issue_outs(k, p):
            r0, c0 = chunk_slices(k)
            pltpu.async_copy(
                rows_bufs[p],
                out_hbm.at[0, :, pl.ds(r0, ROWB), pl.ds(c0, COLB)],
                souts[p],
            )

        def wait_outs(p):
            pltpu.make_async_copy(
                rows_bufs[p],
                out_hbm.at[0, :, pl.ds(0, ROWB), pl.ds(0, COLB)],
                souts[p],
            ).wait()

        def compute(p):
            idx_b, rows_b = idx_bufs[p], rows_bufs[p]

            @plsc.parallel_loop(0, ROWB * qper, unroll=4)
            def _(t):
                ri = t // qper
                q = t % qper
                f = idx_b[ri, pl.ds(q * LANES, LANES)]
                for hp in range(NPAIR):
                    w = plsc.load_gather(tab_v, [f + hp * PADV])
                    lo = plsc.bitcast(lax.shift_left(w, 16), jnp.float32)
                    hi = plsc.bitcast(w & jnp.int32(-65536), jnp.float32)
                    rows_b[2 * hp, ri, pl.ds(q * LANES, LANES)] = lo
                    rows_b[2 * hp + 1, ri, pl.ds(q * LANES, LANES)] = hi

        issue_in(0, 0)

        @pl.loop(0, nchunks // 2)
        def _(g):
            for p in range(2):
                k = g * 2 + p

                @pl.when(k + 1 < nchunks)
                def _():
                    issue_in(k + 1, 1 - p)

                wait_in(p)

                @pl.when(k >= 2)
                def _():
                    wait_outs(p)

                compute(p)
                issue_outs(k, p)

        wait_outs(0)
        wait_outs(1)

    return run(idx2, tab_t)


def kernel(d_mat, embeddings_table):
    b, rows, cols = d_mat.shape
    idx2 = _compute_idx(d_mat.reshape(b * rows, cols))
    tb = lax.bitcast_convert_type(
        embeddings_table.T.astype(jnp.bfloat16), jnp.uint16
    ).astype(jnp.uint32)
    packed = (tb[0::2, :] | (tb[1::2, :] << 16)).astype(jnp.int32)
    tab_t = (
        jnp.zeros((NPAIR, PADV), jnp.int32)
        .at[:, : N_POS_EMB_K + 1]
        .set(packed)
        .reshape(-1)
    )
    out = _sc_gather(idx2, tab_t)
    return out.transpose(0, 2, 3, 1)
